# Initial kernel scaffold; baseline (speedup 1.0000x reference)
import jax
import jax.numpy as jnp
from jax.experimental import pallas as pl


def kernel(word_inputs, feature_inputs, word_seq_lengths, char_inputs, char_seq_lengths, char_seq_recover, batch_word_text, word_table, feat_table0, feat_table1):
    raise NotImplementedError("write your pallas kernel here")


# speedup vs baseline: 1.7649x; 1.7649x over previous
import functools

import jax
import jax.numpy as jnp
from jax import lax
from jax.experimental import pallas as pl
from jax.experimental.pallas import tpu as pltpu
from jax.experimental.pallas import tpu_sc as plsc

_B = 1024
_L = 200
_EMB = 64
_FEAT_EMB = 16
_OUT_D = _EMB + 2 * _FEAT_EMB
_N = _B * _L

_INFO = plsc.get_sparse_core_info()
_NC = _INFO.num_cores
_NS = _INFO.num_subcores
_NW = _NC * _NS

_G = 128
_ROWS_PER_W = _N // _NW
_CHUNK_BLOCKS = 5
_CHUNK_ROWS = _CHUNK_BLOCKS * _G
_NCHUNK = _ROWS_PER_W // _CHUNK_ROWS

_mesh = plsc.VectorSubcoreMesh(core_axis_name="c", subcore_axis_name="s")


@functools.partial(
    pl.kernel,
    mesh=_mesh,
    compiler_params=pltpu.CompilerParams(use_tc_tiling_on_sc=False),
    out_type=jax.ShapeDtypeStruct((_N, _OUT_D), jnp.float32),
    scratch_types=[
        pltpu.VMEM((_CHUNK_ROWS,), jnp.int32),
        pltpu.VMEM((_CHUNK_ROWS,), jnp.int32),
        pltpu.VMEM((_CHUNK_ROWS,), jnp.int32),
        pltpu.VMEM((_CHUNK_ROWS, _EMB), jnp.float32),
        pltpu.VMEM((_CHUNK_ROWS, _FEAT_EMB), jnp.float32),
        pltpu.VMEM((_CHUNK_ROWS, _FEAT_EMB), jnp.float32),
        pltpu.SemaphoreType.DMA,
    ],
)
def _embed(widx_hbm, f0idx_hbm, f1idx_hbm, wtab_hbm, f0tab_hbm, f1tab_hbm,
           out_hbm, widx_v, f0idx_v, f1idx_v, w_v, f0_v, f1_v, sem):
    wid = lax.axis_index("s") * _NC + lax.axis_index("c")

    def chunk_body(ci):
        row0 = wid * _ROWS_PER_W + ci * _CHUNK_ROWS
        pltpu.sync_copy(widx_hbm.at[pl.ds(row0, _CHUNK_ROWS)], widx_v)
        pltpu.sync_copy(f0idx_hbm.at[pl.ds(row0, _CHUNK_ROWS)], f0idx_v)
        pltpu.sync_copy(f1idx_hbm.at[pl.ds(row0, _CHUNK_ROWS)], f1idx_v)
        copies = []
        for g in range(_CHUNK_BLOCKS):
            rows = pl.ds(g * _G, _G)
            copies.append(pltpu.async_copy(
                wtab_hbm.at[widx_v.at[rows]], w_v.at[rows], sem))
            copies.append(pltpu.async_copy(
                f0tab_hbm.at[f0idx_v.at[rows]], f0_v.at[rows], sem))
            copies.append(pltpu.async_copy(
                f1tab_hbm.at[f1idx_v.at[rows]], f1_v.at[rows], sem))
        for c in copies:
            c.wait()
        out_rows = pl.ds(row0, _CHUNK_ROWS)
        pltpu.sync_copy(w_v, out_hbm.at[out_rows, pl.ds(0, _EMB)])
        pltpu.sync_copy(f0_v, out_hbm.at[out_rows, pl.ds(_EMB, _FEAT_EMB)])
        pltpu.sync_copy(
            f1_v, out_hbm.at[out_rows, pl.ds(_EMB + _FEAT_EMB, _FEAT_EMB)])

    pl.loop(0, _NCHUNK)(chunk_body)


def kernel(word_inputs, feature_inputs, word_seq_lengths, char_inputs,
           char_seq_lengths, char_seq_recover, batch_word_text,
           word_table, feat_table0, feat_table1):
    widx = word_inputs.astype(jnp.int32).reshape(_N)
    f0idx = feature_inputs[0].astype(jnp.int32).reshape(_N)
    f1idx = feature_inputs[1].astype(jnp.int32).reshape(_N)
    out = _embed(widx, f0idx, f1idx, word_table, feat_table0, feat_table1)
    return out.reshape(_B, _L, _OUT_D)


# speedup vs baseline: 2.0754x; 1.1760x over previous
import functools

import jax
import jax.numpy as jnp
from jax import lax
from jax.experimental import pallas as pl
from jax.experimental.pallas import tpu as pltpu
from jax.experimental.pallas import tpu_sc as plsc

_B = 1024
_L = 200
_EMB = 64
_FEAT_EMB = 16
_OUT_D = _EMB + 2 * _FEAT_EMB
_N = _B * _L

_INFO = plsc.get_sparse_core_info()
_NC = _INFO.num_cores
_NS = _INFO.num_subcores
_NW = _NC * _NS

_B_PER_W = _B // _NW
_CHUNK_B = 2
_CHUNK_ROWS = _CHUNK_B * _L
_NCHUNK = _B_PER_W // _CHUNK_B
_G = 80
_NG = _CHUNK_ROWS // _G

_FVOCAB = 50

_mesh = plsc.VectorSubcoreMesh(core_axis_name="c", subcore_axis_name="s")


@functools.partial(
    pl.kernel,
    mesh=_mesh,
    compiler_params=pltpu.CompilerParams(use_tc_tiling_on_sc=False),
    out_type=jax.ShapeDtypeStruct((_B, _L, _OUT_D), jnp.float32),
    scratch_types=[
        pltpu.VMEM((2, _CHUNK_ROWS), jnp.int32),
        pltpu.VMEM((2, _CHUNK_ROWS), jnp.int32),
        pltpu.VMEM((2, _CHUNK_ROWS), jnp.int32),
        pltpu.VMEM((2, _CHUNK_ROWS, _EMB), jnp.float32),
        pltpu.VMEM((2, _CHUNK_ROWS, _FEAT_EMB), jnp.float32),
        pltpu.VMEM((2, _CHUNK_ROWS, _FEAT_EMB), jnp.float32),
        pltpu.SemaphoreType.DMA((2,)),
        pltpu.SemaphoreType.DMA((2,)),
    ],
)
def _embed(widx_hbm, f0idx_hbm, f1idx_hbm, wtab_hbm, f0tab_hbm, f1tab_hbm,
           out_hbm, widx_v, f0idx_v, f1idx_v, w_v, f0_v, f1_v, gsem, wsem):
    wid = lax.axis_index("s") * _NC + lax.axis_index("c")
    row_base = wid * _B_PER_W * _L
    b_base = wid * _B_PER_W

    def issue_chunk(ci, buf):
        row0 = row_base + ci * _CHUNK_ROWS
        pltpu.sync_copy(widx_hbm.at[pl.ds(row0, _CHUNK_ROWS)], widx_v.at[buf])
        pltpu.sync_copy(f0idx_hbm.at[pl.ds(row0, _CHUNK_ROWS)], f0idx_v.at[buf])
        pltpu.sync_copy(f1idx_hbm.at[pl.ds(row0, _CHUNK_ROWS)], f1idx_v.at[buf])
        for g in range(_NG):
            rows = pl.ds(g * _G, _G)
            pltpu.async_copy(
                wtab_hbm.at[widx_v.at[buf, rows]], w_v.at[buf, rows],
                gsem.at[buf])
            pltpu.async_copy(
                f0tab_hbm.at[f0idx_v.at[buf, rows]], f0_v.at[buf, rows],
                gsem.at[buf])
            pltpu.async_copy(
                f1tab_hbm.at[f1idx_v.at[buf, rows]], f1_v.at[buf, rows],
                gsem.at[buf])

    def wait_gathers(buf):
        for g in range(_NG):
            rows = pl.ds(g * _G, _G)
            pltpu.make_async_copy(
                wtab_hbm.at[widx_v.at[buf, rows]], w_v.at[buf, rows],
                gsem.at[buf]).wait()
            pltpu.make_async_copy(
                f0tab_hbm.at[f0idx_v.at[buf, rows]], f0_v.at[buf, rows],
                gsem.at[buf]).wait()
            pltpu.make_async_copy(
                f1tab_hbm.at[f1idx_v.at[buf, rows]], f1_v.at[buf, rows],
                gsem.at[buf]).wait()

    def issue_writes(ci, buf):
        for b in range(_CHUNK_B):
            rows = pl.ds(b * _L, _L)
            bo = b_base + ci * _CHUNK_B + b
            pltpu.async_copy(
                w_v.at[buf, rows], out_hbm.at[bo, slice(None), pl.ds(0, _EMB)],
                wsem.at[buf])
            pltpu.async_copy(
                f0_v.at[buf, rows],
                out_hbm.at[bo, slice(None), pl.ds(_EMB, _FEAT_EMB)],
                wsem.at[buf])
            pltpu.async_copy(
                f1_v.at[buf, rows],
                out_hbm.at[bo, slice(None), pl.ds(_EMB + _FEAT_EMB, _FEAT_EMB)],
                wsem.at[buf])

    def wait_writes(ci, buf):
        for b in range(_CHUNK_B):
            rows = pl.ds(b * _L, _L)
            bo = b_base + ci * _CHUNK_B + b
            pltpu.make_async_copy(
                w_v.at[buf, rows], out_hbm.at[bo, slice(None), pl.ds(0, _EMB)],
                wsem.at[buf]).wait()
            pltpu.make_async_copy(
                f0_v.at[buf, rows],
                out_hbm.at[bo, slice(None), pl.ds(_EMB, _FEAT_EMB)],
                wsem.at[buf]).wait()
            pltpu.make_async_copy(
                f1_v.at[buf, rows],
                out_hbm.at[bo, slice(None), pl.ds(_EMB + _FEAT_EMB, _FEAT_EMB)],
                wsem.at[buf]).wait()

    issue_chunk(0, 0)

    def body(ci):
        buf = lax.rem(ci, 2)
        nxt = lax.rem(ci + 1, 2)

        @pl.when(ci >= 1)
        def _():
            wait_writes(ci - 1, nxt)

        @pl.when(ci + 1 < _NCHUNK)
        def _():
            issue_chunk(ci + 1, nxt)

        wait_gathers(buf)
        issue_writes(ci, buf)

    pl.loop(0, _NCHUNK)(body)
    wait_writes(_NCHUNK - 1, (_NCHUNK - 1) % 2)


def kernel(word_inputs, feature_inputs, word_seq_lengths, char_inputs,
           char_seq_lengths, char_seq_recover, batch_word_text,
           word_table, feat_table0, feat_table1):
    widx = word_inputs.astype(jnp.int32).reshape(_N)
    bias = (jnp.arange(_N, dtype=jnp.int32) // (_B_PER_W * _L)) * _FVOCAB
    f0idx = feature_inputs[0].astype(jnp.int32).reshape(_N) + bias
    f1idx = feature_inputs[1].astype(jnp.int32).reshape(_N) + bias
    f0rep = jnp.tile(feat_table0, (_NW, 1))
    f1rep = jnp.tile(feat_table1, (_NW, 1))
    return _embed(widx, f0idx, f1idx, word_table, f0rep, f1rep)


# speedup vs baseline: 2.3309x; 1.1231x over previous
import functools

import jax
import jax.numpy as jnp
from jax import lax
from jax.experimental import pallas as pl
from jax.experimental.pallas import tpu as pltpu
from jax.experimental.pallas import tpu_sc as plsc

_B = 1024
_L = 200
_EMB = 64
_FEAT_EMB = 16
_OUT_D = _EMB + 2 * _FEAT_EMB
_N = _B * _L

_INFO = plsc.get_sparse_core_info()
_NC = _INFO.num_cores
_NS = _INFO.num_subcores
_NW = _NC * _NS

_B_PER_W = _B // _NW
_CHUNK_B = 2
_CHUNK_ROWS = _CHUNK_B * _L
_NCHUNK = _B_PER_W // _CHUNK_B
_G = 80
_NG = _CHUNK_ROWS // _G

_FVOCAB = 50

_mesh = plsc.VectorSubcoreMesh(core_axis_name="c", subcore_axis_name="s")


@functools.partial(
    pl.kernel,
    mesh=_mesh,
    compiler_params=pltpu.CompilerParams(use_tc_tiling_on_sc=False),
    out_type=jax.ShapeDtypeStruct((_B, _L, 128), jnp.float32),
    scratch_types=[
        pltpu.VMEM((2, _CHUNK_ROWS), jnp.int32),
        pltpu.VMEM((2, _CHUNK_ROWS), jnp.int32),
        pltpu.VMEM((2, _CHUNK_ROWS), jnp.int32),
        pltpu.VMEM((2, _CHUNK_ROWS, _EMB), jnp.float32),
        pltpu.VMEM((2, _CHUNK_ROWS, _FEAT_EMB), jnp.float32),
        pltpu.VMEM((2, _CHUNK_ROWS, _FEAT_EMB), jnp.float32),
        pltpu.SemaphoreType.DMA((2,)),
        pltpu.SemaphoreType.DMA((2,)),
    ],
)
def _embed(widx_hbm, f0idx_hbm, f1idx_hbm, wtab_hbm, f0tab_hbm, f1tab_hbm,
           out_hbm, widx_v, f0idx_v, f1idx_v, w_v, f0_v, f1_v, gsem, wsem):
    wid = lax.axis_index("s") * _NC + lax.axis_index("c")
    row_base = wid * _B_PER_W * _L
    b_base = wid * _B_PER_W

    def issue_chunk(ci, buf):
        row0 = row_base + ci * _CHUNK_ROWS
        pltpu.sync_copy(widx_hbm.at[pl.ds(row0, _CHUNK_ROWS)], widx_v.at[buf])
        pltpu.sync_copy(f0idx_hbm.at[pl.ds(row0, _CHUNK_ROWS)], f0idx_v.at[buf])
        pltpu.sync_copy(f1idx_hbm.at[pl.ds(row0, _CHUNK_ROWS)], f1idx_v.at[buf])
        for g in range(_NG):
            rows = pl.ds(g * _G, _G)
            pltpu.async_copy(
                wtab_hbm.at[widx_v.at[buf, rows]], w_v.at[buf, rows],
                gsem.at[buf])
            pltpu.async_copy(
                f0tab_hbm.at[f0idx_v.at[buf, rows]], f0_v.at[buf, rows],
                gsem.at[buf])
            pltpu.async_copy(
                f1tab_hbm.at[f1idx_v.at[buf, rows]], f1_v.at[buf, rows],
                gsem.at[buf])

    def wait_gathers(buf):
        for g in range(_NG):
            rows = pl.ds(g * _G, _G)
            pltpu.make_async_copy(
                wtab_hbm.at[widx_v.at[buf, rows]], w_v.at[buf, rows],
                gsem.at[buf]).wait()
            pltpu.make_async_copy(
                f0tab_hbm.at[f0idx_v.at[buf, rows]], f0_v.at[buf, rows],
                gsem.at[buf]).wait()
            pltpu.make_async_copy(
                f1tab_hbm.at[f1idx_v.at[buf, rows]], f1_v.at[buf, rows],
                gsem.at[buf]).wait()

    def issue_writes(ci, buf):
        for b in range(_CHUNK_B):
            rows = pl.ds(b * _L, _L)
            bo = b_base + ci * _CHUNK_B + b
            pltpu.async_copy(
                w_v.at[buf, rows], out_hbm.at[bo, slice(None), pl.ds(0, _EMB)],
                wsem.at[buf])
            pltpu.async_copy(
                f0_v.at[buf, rows],
                out_hbm.at[bo, slice(None), pl.ds(_EMB, _FEAT_EMB)],
                wsem.at[buf])
            pltpu.async_copy(
                f1_v.at[buf, rows],
                out_hbm.at[bo, slice(None), pl.ds(_EMB + _FEAT_EMB, _FEAT_EMB)],
                wsem.at[buf])

    def wait_writes(ci, buf):
        for b in range(_CHUNK_B):
            rows = pl.ds(b * _L, _L)
            bo = b_base + ci * _CHUNK_B + b
            pltpu.make_async_copy(
                w_v.at[buf, rows], out_hbm.at[bo, slice(None), pl.ds(0, _EMB)],
                wsem.at[buf]).wait()
            pltpu.make_async_copy(
                f0_v.at[buf, rows],
                out_hbm.at[bo, slice(None), pl.ds(_EMB, _FEAT_EMB)],
                wsem.at[buf]).wait()
            pltpu.make_async_copy(
                f1_v.at[buf, rows],
                out_hbm.at[bo, slice(None), pl.ds(_EMB + _FEAT_EMB, _FEAT_EMB)],
                wsem.at[buf]).wait()

    issue_chunk(0, 0)

    def body(ci):
        buf = lax.rem(ci, 2)
        nxt = lax.rem(ci + 1, 2)

        @pl.when(ci >= 1)
        def _():
            wait_writes(ci - 1, nxt)

        @pl.when(ci + 1 < _NCHUNK)
        def _():
            issue_chunk(ci + 1, nxt)

        wait_gathers(buf)
        issue_writes(ci, buf)

    pl.loop(0, _NCHUNK)(body)
    wait_writes(_NCHUNK - 1, (_NCHUNK - 1) % 2)


def kernel(word_inputs, feature_inputs, word_seq_lengths, char_inputs,
           char_seq_lengths, char_seq_recover, batch_word_text,
           word_table, feat_table0, feat_table1):
    widx = word_inputs.astype(jnp.int32).reshape(_N)
    bias = (jnp.arange(_N, dtype=jnp.int32) // (_B_PER_W * _L)) * _FVOCAB
    f0idx = feature_inputs[0].astype(jnp.int32).reshape(_N) + bias
    f1idx = feature_inputs[1].astype(jnp.int32).reshape(_N) + bias
    f0rep = jnp.tile(feat_table0, (_NW, 1))
    f1rep = jnp.tile(feat_table1, (_NW, 1))
    out = _embed(widx, f0idx, f1idx, word_table, f0rep, f1rep)
    return out[:, :, :_OUT_D]
